# Initial kernel scaffold; baseline (speedup 1.0000x reference)
import jax
import jax.numpy as jnp
from jax.experimental import pallas as pl


def kernel(inputs, positional_encoding):
    raise NotImplementedError("write your pallas kernel here")


# speedup vs baseline: 1.7205x; 1.7205x over previous
import jax
import jax.numpy as jnp
from jax.experimental import pallas as pl

_BS = 512


def _add_pe_kernel(x_ref, pe_ref, o_ref):
    o_ref[...] = x_ref[...] + pe_ref[...][None, :, :]


def kernel(inputs, positional_encoding):
    b, s, f = inputs.shape
    grid = (s // _BS,)
    return pl.pallas_call(
        _add_pe_kernel,
        grid=grid,
        in_specs=[
            pl.BlockSpec((b, _BS, f), lambda i: (0, i, 0)),
            pl.BlockSpec((_BS, f), lambda i: (i, 0)),
        ],
        out_specs=pl.BlockSpec((b, _BS, f), lambda i: (0, i, 0)),
        out_shape=jax.ShapeDtypeStruct((b, s, f), inputs.dtype),
    )(inputs, positional_encoding[:s])
